# Initial kernel scaffold; baseline (speedup 1.0000x reference)
import jax
import jax.numpy as jnp
from jax.experimental import pallas as pl


def kernel(x, edge_index, edge_attr, batch, W1, b1, W2, b2, W3, b3, fc_w, fc_b):
    raise NotImplementedError("write your pallas kernel here")


# speedup vs baseline: 5.9264x; 5.9264x over previous
import functools

import jax
import jax.numpy as jnp
from jax import lax
from jax.experimental import pallas as pl
from jax.experimental.pallas import tpu as pltpu
from jax.experimental.pallas import tpu_sc as plsc

N = 10000
E = 320000
D = 128
DH = D // 2
NG = 64

NC = 2
NS = 16
NT = NC * NS
EPT = E // NT
K = 80
NCHUNK = EPT // K
ZCH = 80
NZ = N // ZCH

_vmesh = plsc.VectorSubcoreMesh(core_axis_name="c", subcore_axis_name="s")
_sc_params = pltpu.CompilerParams(use_tc_tiling_on_sc=False)


@functools.partial(
    pl.kernel,
    out_type=jax.ShapeDtypeStruct((NC, 1, N), jnp.float32),
    mesh=_vmesh,
    scratch_types=[
        pltpu.VMEM((NCHUNK, K), jnp.int32),
        pltpu.VMEM((NCHUNK, K), jnp.float32),
        pltpu.VMEM((N,), jnp.float32),
        pltpu.VMEM_SHARED((N,), jnp.float32),
        pltpu.SemaphoreType.DMA,
    ],
    compiler_params=_sc_params,
)
def _sc_deg(col_hbm, ew_hbm, out_hbm, col_v, ew_v, zb_v, deg_sp, sem):
    c = lax.axis_index("c")
    s = lax.axis_index("s")
    w = c * NS + s

    @pl.when(s == 0)
    def _():
        zeros16 = jnp.zeros((16,), jnp.float32)

        @pl.loop(0, N // 16)
        def _(j):
            zb_v[pl.ds(pl.multiple_of(j * 16, 16), 16)] = zeros16

        pltpu.sync_copy(zb_v, deg_sp)

    pltpu.sync_copy(col_hbm.at[w], col_v)
    pltpu.sync_copy(ew_hbm.at[w], ew_v)
    plsc.subcore_barrier()

    @pl.loop(0, NCHUNK)
    def _(j):
        pltpu.sync_copy(ew_v.at[j], deg_sp.at[col_v.at[j]], add=True)

    plsc.subcore_barrier()

    @pl.when(s == 0)
    def _():
        pltpu.async_copy(deg_sp, out_hbm.at[c, 0], sem).wait()


_acc_half_type = jax.ShapeDtypeStruct((NC, N, DH), jnp.float32)


@functools.partial(
    pl.kernel,
    out_type=[_acc_half_type, _acc_half_type],
    mesh=_vmesh,
    scratch_types=[
        pltpu.VMEM((NCHUNK, K), jnp.int32),
        pltpu.VMEM((NCHUNK, K), jnp.int32),
        pltpu.VMEM((NCHUNK, K), jnp.float32),
        pltpu.VMEM((K, DH), jnp.float32),
        pltpu.VMEM((ZCH, DH), jnp.float32),
        pltpu.VMEM_SHARED((N, DH), jnp.float32),
        pltpu.SemaphoreType.DMA,
    ],
    compiler_params=_sc_params,
)
def _sc_scatter(ga_hbm, gb_hbm, row_hbm, col_hbm, ew_hbm, outa_hbm, outb_hbm,
                row_v, col_v, ew_v, rows_v, zb_v, acc_sp, sem):
    c = lax.axis_index("c")
    s = lax.axis_index("s")
    w = c * NS + s

    zeros16 = jnp.zeros((16,), jnp.float32)

    @pl.loop(0, ZCH)
    def _(j):
        for f in range(DH // 16):
            zb_v[j, pl.ds(f * 16, 16)] = zeros16

    pltpu.sync_copy(row_hbm.at[w], row_v)
    pltpu.sync_copy(col_hbm.at[w], col_v)
    pltpu.sync_copy(ew_hbm.at[w], ew_v)

    for p, (g_hbm, out_hbm) in enumerate(((ga_hbm, outa_hbm),
                                          (gb_hbm, outb_hbm))):
        @pl.loop(0, pl.cdiv(NZ, NS))
        def _(t):
            m = s + t * NS

            @pl.when(m < NZ)
            def _():
                off = pl.multiple_of(m * ZCH, 8)
                pltpu.sync_copy(zb_v, acc_sp.at[pl.ds(off, ZCH)])

        plsc.subcore_barrier()

        @pl.loop(0, NCHUNK)
        def _(j):
            pltpu.async_copy(g_hbm.at[row_v.at[j]], rows_v, sem).wait()

            @pl.loop(0, K, step=16)
            def _(k0):
                ew16 = ew_v[j, pl.ds(k0, 16)]
                for u in range(16):
                    sc = ew16[u]
                    for f in range(DH // 16):
                        sl = pl.ds(f * 16, 16)
                        rows_v[k0 + u, sl] = rows_v[k0 + u, sl] * sc

            pltpu.sync_copy(rows_v, acc_sp.at[col_v.at[j]], add=True)

        plsc.subcore_barrier()

        @pl.loop(0, pl.cdiv(NZ, NS))
        def _(t):
            m = s + t * NS

            @pl.when(m < NZ)
            def _():
                off = pl.multiple_of(m * ZCH, 8)
                pltpu.async_copy(acc_sp.at[pl.ds(off, ZCH)],
                                 out_hbm.at[c, pl.ds(off, ZCH)], sem).wait()

        plsc.subcore_barrier()


_BLK = 1000
_GRID = N // _BLK


def _mm_body(x_ref, w_ref, o_ref):
    o_ref[...] = jnp.dot(x_ref[...], w_ref[...],
                         preferred_element_type=jnp.float32)


def _tc_matmul(x, w):
    return pl.pallas_call(
        _mm_body,
        grid=(_GRID,),
        in_specs=[pl.BlockSpec((_BLK, D), lambda i: (i, 0)),
                  pl.BlockSpec((D, D), lambda i: (0, 0))],
        out_specs=pl.BlockSpec((_BLK, D), lambda i: (i, 0)),
        out_shape=jax.ShapeDtypeStruct((N, D), jnp.float32),
    )(x, w)


def _scale_body(degp_ref, h_ref, dinv_ref, ga_ref, gb_ref):
    deg = degp_ref[0] + degp_ref[1] + 1.0
    dv = lax.rsqrt(deg)
    dinv_ref[...] = dv
    g = dv * h_ref[...]
    ga_ref[...] = g[:, :DH]
    gb_ref[...] = g[:, DH:]


def _tc_scale(degp, h):
    return pl.pallas_call(
        _scale_body,
        grid=(_GRID,),
        in_specs=[pl.BlockSpec((NC, _BLK, 1), lambda i: (0, i, 0)),
                  pl.BlockSpec((_BLK, D), lambda i: (i, 0))],
        out_specs=[pl.BlockSpec((_BLK, 1), lambda i: (i, 0)),
                   pl.BlockSpec((_BLK, DH), lambda i: (i, 0)),
                   pl.BlockSpec((_BLK, DH), lambda i: (i, 0))],
        out_shape=[jax.ShapeDtypeStruct((N, 1), jnp.float32),
                   jax.ShapeDtypeStruct((N, DH), jnp.float32),
                   jax.ShapeDtypeStruct((N, DH), jnp.float32)],
    )(degp, h)


def _combine(acca_ref, accb_ref, ga_ref, gb_ref):
    ta = acca_ref[0] + acca_ref[1] + ga_ref[...]
    tb = accb_ref[0] + accb_ref[1] + gb_ref[...]
    return jnp.concatenate([ta, tb], axis=1)


def _post_mm_body(acca_ref, accb_ref, ga_ref, gb_ref, dinv_ref, b_ref, w_ref,
                  oa_ref, ob_ref):
    dv = dinv_ref[...]
    t = _combine(acca_ref, accb_ref, ga_ref, gb_ref)
    t = jnp.maximum(dv * t + b_ref[...], 0.0)
    r = dv * jnp.dot(t, w_ref[...], preferred_element_type=jnp.float32)
    oa_ref[...] = r[:, :DH]
    ob_ref[...] = r[:, DH:]


def _tc_post_mm(acca, accb, ga, gb, dinv, b, w):
    return pl.pallas_call(
        _post_mm_body,
        grid=(_GRID,),
        in_specs=[pl.BlockSpec((NC, _BLK, DH), lambda i: (0, i, 0)),
                  pl.BlockSpec((NC, _BLK, DH), lambda i: (0, i, 0)),
                  pl.BlockSpec((_BLK, DH), lambda i: (i, 0)),
                  pl.BlockSpec((_BLK, DH), lambda i: (i, 0)),
                  pl.BlockSpec((_BLK, 1), lambda i: (i, 0)),
                  pl.BlockSpec((1, D), lambda i: (0, 0)),
                  pl.BlockSpec((D, D), lambda i: (0, 0))],
        out_specs=[pl.BlockSpec((_BLK, DH), lambda i: (i, 0)),
                   pl.BlockSpec((_BLK, DH), lambda i: (i, 0))],
        out_shape=[jax.ShapeDtypeStruct((N, DH), jnp.float32),
                   jax.ShapeDtypeStruct((N, DH), jnp.float32)],
    )(acca, accb, ga, gb, dinv, b, w)


def _final_body(acca_ref, accb_ref, ga_ref, gb_ref, dinv_ref, b_ref,
                batch_ref, fcw_ref, fcb_ref, o_ref, sums_ref, cnts_ref):
    i = pl.program_id(0)

    @pl.when(i == 0)
    def _():
        sums_ref[...] = jnp.zeros_like(sums_ref)
        cnts_ref[...] = jnp.zeros_like(cnts_ref)

    dv = dinv_ref[...]
    t = _combine(acca_ref, accb_ref, ga_ref, gb_ref)
    h3r = jnp.maximum(dv * t + b_ref[...], 0.0)
    gid = lax.broadcasted_iota(jnp.int32, (_BLK, NG), 1)
    oh = (batch_ref[...] == gid).astype(jnp.float32)
    dn = (((0,), (0,)), ((), ()))
    sums_ref[...] += lax.dot_general(oh, h3r, dn,
                                     preferred_element_type=jnp.float32)
    cnts_ref[...] += lax.dot_general(oh, jnp.ones_like(h3r), dn,
                                     preferred_element_type=jnp.float32)

    @pl.when(i == _GRID - 1)
    def _():
        pooled = sums_ref[...] / jnp.maximum(cnts_ref[...], 1.0)
        z = jnp.dot(pooled, fcw_ref[...],
                    preferred_element_type=jnp.float32) + fcb_ref[...]
        o_ref[...] = 1.0 / (1.0 + jnp.exp(-z))


def _tc_final(acca, accb, ga, gb, dinv, b, batch2, fc_w, fc_b):
    return pl.pallas_call(
        _final_body,
        grid=(_GRID,),
        in_specs=[pl.BlockSpec((NC, _BLK, DH), lambda i: (0, i, 0)),
                  pl.BlockSpec((NC, _BLK, DH), lambda i: (0, i, 0)),
                  pl.BlockSpec((_BLK, DH), lambda i: (i, 0)),
                  pl.BlockSpec((_BLK, DH), lambda i: (i, 0)),
                  pl.BlockSpec((_BLK, 1), lambda i: (i, 0)),
                  pl.BlockSpec((1, D), lambda i: (0, 0)),
                  pl.BlockSpec((_BLK, 1), lambda i: (i, 0)),
                  pl.BlockSpec((D, 1), lambda i: (0, 0)),
                  pl.BlockSpec((1, 1), lambda i: (0, 0))],
        out_specs=pl.BlockSpec((NG, 1), lambda i: (0, 0)),
        out_shape=jax.ShapeDtypeStruct((NG, 1), jnp.float32),
        scratch_shapes=[pltpu.VMEM((NG, D), jnp.float32),
                        pltpu.VMEM((NG, D), jnp.float32)],
    )(acca, accb, ga, gb, dinv, b, batch2, fc_w, fc_b)


def kernel(x, edge_index, edge_attr, batch, W1, b1, W2, b2, W3, b3, fc_w, fc_b):
    row = edge_index[0].astype(jnp.int32).reshape(NT, NCHUNK, K)
    col = edge_index[1].astype(jnp.int32).reshape(NT, NCHUNK, K)
    ew = edge_attr.astype(jnp.float32).reshape(NT, NCHUNK, K)
    batch2 = batch.astype(jnp.int32).reshape(N, 1)
    b1r = b1.reshape(1, D)
    b2r = b2.reshape(1, D)
    b3r = b3.reshape(1, D)
    fcb = fc_b.reshape(1, 1)

    degp = _sc_deg(col, ew)
    h1 = _tc_matmul(x, W1)
    dinv, g1a, g1b = _tc_scale(degp.reshape(NC, N, 1), h1)

    a1a, a1b = _sc_scatter(g1a, g1b, row, col, ew)
    g2a, g2b = _tc_post_mm(a1a, a1b, g1a, g1b, dinv, b1r, W2)
    a2a, a2b = _sc_scatter(g2a, g2b, row, col, ew)
    g3a, g3b = _tc_post_mm(a2a, a2b, g2a, g2b, dinv, b2r, W3)
    a3a, a3b = _sc_scatter(g3a, g3b, row, col, ew)
    return _tc_final(a3a, a3b, g3a, g3b, dinv, b3r, batch2, fc_w, fcb)


# speedup vs baseline: 8.4154x; 1.4200x over previous
import functools

import jax
import jax.numpy as jnp
from jax import lax
from jax.experimental import pallas as pl
from jax.experimental.pallas import tpu as pltpu
from jax.experimental.pallas import tpu_sc as plsc

N = 10000
E = 320000
D = 128
DH = D // 2
NG = 64

NC = 2
NS = 16
NT = NC * NS
EPT = E // NT
K = 80
NCHUNK = EPT // K
ZCH = 80
NZ = N // ZCH

_vmesh = plsc.VectorSubcoreMesh(core_axis_name="c", subcore_axis_name="s")
_sc_params = pltpu.CompilerParams(use_tc_tiling_on_sc=False)


@functools.partial(
    pl.kernel,
    out_type=jax.ShapeDtypeStruct((NC, 1, N), jnp.float32),
    mesh=_vmesh,
    scratch_types=[
        pltpu.VMEM((NCHUNK, K), jnp.int32),
        pltpu.VMEM((NCHUNK, K), jnp.float32),
        pltpu.VMEM((N,), jnp.float32),
        pltpu.VMEM_SHARED((N,), jnp.float32),
        pltpu.SemaphoreType.DMA,
    ],
    compiler_params=_sc_params,
)
def _sc_deg(col_hbm, ew_hbm, out_hbm, col_v, ew_v, zb_v, deg_sp, sem):
    c = lax.axis_index("c")
    s = lax.axis_index("s")
    w = c * NS + s

    @pl.when(s == 0)
    def _():
        zeros16 = jnp.zeros((16,), jnp.float32)

        @pl.loop(0, N // 16)
        def _(j):
            zb_v[pl.ds(pl.multiple_of(j * 16, 16), 16)] = zeros16

        pltpu.sync_copy(zb_v, deg_sp)

    pltpu.sync_copy(col_hbm.at[w], col_v)
    pltpu.sync_copy(ew_hbm.at[w], ew_v)
    plsc.subcore_barrier()

    @pl.loop(0, NCHUNK)
    def _(j):
        pltpu.sync_copy(ew_v.at[j], deg_sp.at[col_v.at[j]], add=True)

    plsc.subcore_barrier()

    @pl.when(s == 0)
    def _():
        pltpu.async_copy(deg_sp, out_hbm.at[c, 0], sem).wait()


_acc_half_type = jax.ShapeDtypeStruct((NC, N, DH), jnp.float32)


@functools.partial(
    pl.kernel,
    out_type=[_acc_half_type, _acc_half_type],
    mesh=_vmesh,
    scratch_types=[
        pltpu.VMEM((NCHUNK, K), jnp.int32),
        pltpu.VMEM((NCHUNK, K), jnp.int32),
        pltpu.VMEM((NCHUNK, K), jnp.float32),
        pltpu.VMEM((K, DH), jnp.float32),
        pltpu.VMEM((K, DH), jnp.float32),
        pltpu.VMEM((ZCH, DH), jnp.float32),
        pltpu.VMEM_SHARED((N, DH), jnp.float32),
        pltpu.SemaphoreType.DMA,
        pltpu.SemaphoreType.DMA,
        pltpu.SemaphoreType.DMA,
    ],
    compiler_params=_sc_params,
)
def _sc_scatter(ga_hbm, gb_hbm, row_hbm, col_hbm, ew_hbm, outa_hbm, outb_hbm,
                row_v, col_v, ew_v, rows0_v, rows1_v, zb_v, acc_sp,
                sem, sem0, sem1):
    c = lax.axis_index("c")
    s = lax.axis_index("s")
    w = c * NS + s

    zeros16 = jnp.zeros((16,), jnp.float32)

    @pl.loop(0, ZCH)
    def _(j):
        for f in range(DH // 16):
            zb_v[j, pl.ds(f * 16, 16)] = zeros16

    pltpu.sync_copy(row_hbm.at[w], row_v)
    pltpu.sync_copy(col_hbm.at[w], col_v)
    pltpu.sync_copy(ew_hbm.at[w], ew_v)

    for p, (g_hbm, out_hbm) in enumerate(((ga_hbm, outa_hbm),
                                          (gb_hbm, outb_hbm))):
        @pl.loop(0, pl.cdiv(NZ, NS))
        def _(t):
            m = s + t * NS

            @pl.when(m < NZ)
            def _():
                off = pl.multiple_of(m * ZCH, 8)
                pltpu.sync_copy(zb_v, acc_sp.at[pl.ds(off, ZCH)])

        plsc.subcore_barrier()

        def _wait(buf, sg, j):
            pltpu.make_async_copy(g_hbm.at[row_v.at[j]], buf, sg).wait()

        def _do(j, buf):
            @pl.loop(0, K, step=16)
            def _(k0):
                ew16 = ew_v[j, pl.ds(k0, 16)]
                for u in range(16):
                    sc = ew16[u]
                    for f in range(DH // 16):
                        sl = pl.ds(f * 16, 16)
                        buf[k0 + u, sl] = buf[k0 + u, sl] * sc

            pltpu.sync_copy(buf, acc_sp.at[col_v.at[j]], add=True)

        pltpu.async_copy(g_hbm.at[row_v.at[0]], rows0_v, sem0)

        @pl.loop(0, (NCHUNK - 1) // 2)
        def _(jj):
            j0 = jj * 2
            _wait(rows0_v, sem0, j0)
            pltpu.async_copy(g_hbm.at[row_v.at[j0 + 1]], rows1_v, sem1)
            _do(j0, rows0_v)
            _wait(rows1_v, sem1, j0 + 1)
            pltpu.async_copy(g_hbm.at[row_v.at[j0 + 2]], rows0_v, sem0)
            _do(j0 + 1, rows1_v)

        _wait(rows0_v, sem0, NCHUNK - 1)
        _do(NCHUNK - 1, rows0_v)

        plsc.subcore_barrier()

        @pl.loop(0, pl.cdiv(NZ, NS))
        def _(t):
            m = s + t * NS

            @pl.when(m < NZ)
            def _():
                off = pl.multiple_of(m * ZCH, 8)
                pltpu.async_copy(acc_sp.at[pl.ds(off, ZCH)],
                                 out_hbm.at[c, pl.ds(off, ZCH)], sem).wait()

        plsc.subcore_barrier()


_BLK = 1000
_GRID = N // _BLK


def _mm_body(x_ref, w_ref, o_ref):
    o_ref[...] = jnp.dot(x_ref[...], w_ref[...],
                         preferred_element_type=jnp.float32)


def _tc_matmul(x, w):
    return pl.pallas_call(
        _mm_body,
        grid=(_GRID,),
        in_specs=[pl.BlockSpec((_BLK, D), lambda i: (i, 0)),
                  pl.BlockSpec((D, D), lambda i: (0, 0))],
        out_specs=pl.BlockSpec((_BLK, D), lambda i: (i, 0)),
        out_shape=jax.ShapeDtypeStruct((N, D), jnp.float32),
    )(x, w)


def _scale_body(degp_ref, h_ref, dinv_ref, ga_ref, gb_ref):
    deg = degp_ref[0] + degp_ref[1] + 1.0
    dv = lax.rsqrt(deg)
    dinv_ref[...] = dv
    g = dv * h_ref[...]
    ga_ref[...] = g[:, :DH]
    gb_ref[...] = g[:, DH:]


def _tc_scale(degp, h):
    return pl.pallas_call(
        _scale_body,
        grid=(_GRID,),
        in_specs=[pl.BlockSpec((NC, _BLK, 1), lambda i: (0, i, 0)),
                  pl.BlockSpec((_BLK, D), lambda i: (i, 0))],
        out_specs=[pl.BlockSpec((_BLK, 1), lambda i: (i, 0)),
                   pl.BlockSpec((_BLK, DH), lambda i: (i, 0)),
                   pl.BlockSpec((_BLK, DH), lambda i: (i, 0))],
        out_shape=[jax.ShapeDtypeStruct((N, 1), jnp.float32),
                   jax.ShapeDtypeStruct((N, DH), jnp.float32),
                   jax.ShapeDtypeStruct((N, DH), jnp.float32)],
    )(degp, h)


def _combine(acca_ref, accb_ref, ga_ref, gb_ref):
    ta = acca_ref[0] + acca_ref[1] + ga_ref[...]
    tb = accb_ref[0] + accb_ref[1] + gb_ref[...]
    return jnp.concatenate([ta, tb], axis=1)


def _post_mm_body(acca_ref, accb_ref, ga_ref, gb_ref, dinv_ref, b_ref, w_ref,
                  oa_ref, ob_ref):
    dv = dinv_ref[...]
    t = _combine(acca_ref, accb_ref, ga_ref, gb_ref)
    t = jnp.maximum(dv * t + b_ref[...], 0.0)
    r = dv * jnp.dot(t, w_ref[...], preferred_element_type=jnp.float32)
    oa_ref[...] = r[:, :DH]
    ob_ref[...] = r[:, DH:]


def _tc_post_mm(acca, accb, ga, gb, dinv, b, w):
    return pl.pallas_call(
        _post_mm_body,
        grid=(_GRID,),
        in_specs=[pl.BlockSpec((NC, _BLK, DH), lambda i: (0, i, 0)),
                  pl.BlockSpec((NC, _BLK, DH), lambda i: (0, i, 0)),
                  pl.BlockSpec((_BLK, DH), lambda i: (i, 0)),
                  pl.BlockSpec((_BLK, DH), lambda i: (i, 0)),
                  pl.BlockSpec((_BLK, 1), lambda i: (i, 0)),
                  pl.BlockSpec((1, D), lambda i: (0, 0)),
                  pl.BlockSpec((D, D), lambda i: (0, 0))],
        out_specs=[pl.BlockSpec((_BLK, DH), lambda i: (i, 0)),
                   pl.BlockSpec((_BLK, DH), lambda i: (i, 0))],
        out_shape=[jax.ShapeDtypeStruct((N, DH), jnp.float32),
                   jax.ShapeDtypeStruct((N, DH), jnp.float32)],
    )(acca, accb, ga, gb, dinv, b, w)


def _final_body(acca_ref, accb_ref, ga_ref, gb_ref, dinv_ref, b_ref,
                batch_ref, fcw_ref, fcb_ref, o_ref, sums_ref, cnts_ref):
    i = pl.program_id(0)

    @pl.when(i == 0)
    def _():
        sums_ref[...] = jnp.zeros_like(sums_ref)
        cnts_ref[...] = jnp.zeros_like(cnts_ref)

    dv = dinv_ref[...]
    t = _combine(acca_ref, accb_ref, ga_ref, gb_ref)
    h3r = jnp.maximum(dv * t + b_ref[...], 0.0)
    gid = lax.broadcasted_iota(jnp.int32, (_BLK, NG), 1)
    oh = (batch_ref[...] == gid).astype(jnp.float32)
    dn = (((0,), (0,)), ((), ()))
    sums_ref[...] += lax.dot_general(oh, h3r, dn,
                                     preferred_element_type=jnp.float32)
    cnts_ref[...] += lax.dot_general(oh, jnp.ones_like(h3r), dn,
                                     preferred_element_type=jnp.float32)

    @pl.when(i == _GRID - 1)
    def _():
        pooled = sums_ref[...] / jnp.maximum(cnts_ref[...], 1.0)
        z = jnp.dot(pooled, fcw_ref[...],
                    preferred_element_type=jnp.float32) + fcb_ref[...]
        o_ref[...] = 1.0 / (1.0 + jnp.exp(-z))


def _tc_final(acca, accb, ga, gb, dinv, b, batch2, fc_w, fc_b):
    return pl.pallas_call(
        _final_body,
        grid=(_GRID,),
        in_specs=[pl.BlockSpec((NC, _BLK, DH), lambda i: (0, i, 0)),
                  pl.BlockSpec((NC, _BLK, DH), lambda i: (0, i, 0)),
                  pl.BlockSpec((_BLK, DH), lambda i: (i, 0)),
                  pl.BlockSpec((_BLK, DH), lambda i: (i, 0)),
                  pl.BlockSpec((_BLK, 1), lambda i: (i, 0)),
                  pl.BlockSpec((1, D), lambda i: (0, 0)),
                  pl.BlockSpec((_BLK, 1), lambda i: (i, 0)),
                  pl.BlockSpec((D, 1), lambda i: (0, 0)),
                  pl.BlockSpec((1, 1), lambda i: (0, 0))],
        out_specs=pl.BlockSpec((NG, 1), lambda i: (0, 0)),
        out_shape=jax.ShapeDtypeStruct((NG, 1), jnp.float32),
        scratch_shapes=[pltpu.VMEM((NG, D), jnp.float32),
                        pltpu.VMEM((NG, D), jnp.float32)],
    )(acca, accb, ga, gb, dinv, b, batch2, fc_w, fc_b)


def kernel(x, edge_index, edge_attr, batch, W1, b1, W2, b2, W3, b3, fc_w, fc_b):
    row = edge_index[0].astype(jnp.int32).reshape(NT, NCHUNK, K)
    col = edge_index[1].astype(jnp.int32).reshape(NT, NCHUNK, K)
    ew = edge_attr.astype(jnp.float32).reshape(NT, NCHUNK, K)
    batch2 = batch.astype(jnp.int32).reshape(N, 1)
    b1r = b1.reshape(1, D)
    b2r = b2.reshape(1, D)
    b3r = b3.reshape(1, D)
    fcb = fc_b.reshape(1, 1)

    degp = _sc_deg(col, ew)
    h1 = _tc_matmul(x, W1)
    dinv, g1a, g1b = _tc_scale(degp.reshape(NC, N, 1), h1)

    a1a, a1b = _sc_scatter(g1a, g1b, row, col, ew)
    g2a, g2b = _tc_post_mm(a1a, a1b, g1a, g1b, dinv, b1r, W2)
    a2a, a2b = _sc_scatter(g2a, g2b, row, col, ew)
    g3a, g3b = _tc_post_mm(a2a, a2b, g2a, g2b, dinv, b2r, W3)
    a3a, a3b = _sc_scatter(g3a, g3b, row, col, ew)
    return _tc_final(a3a, a3b, g3a, g3b, dinv, b3r, batch2, fc_w, fcb)


# speedup vs baseline: 13.5086x; 1.6052x over previous
import functools

import jax
import jax.numpy as jnp
from jax import lax
from jax.experimental import pallas as pl
from jax.experimental.pallas import tpu as pltpu
from jax.experimental.pallas import tpu_sc as plsc

N = 10000
E = 320000
D = 128
DH = D // 2
NG = 64

NC = 2
NS = 16
NT = NC * NS
EPT = E // NT
K = 80
NCHUNK = EPT // K
ZCH = 80
NZ = N // ZCH

_vmesh = plsc.VectorSubcoreMesh(core_axis_name="c", subcore_axis_name="s")
_sc_params = pltpu.CompilerParams(use_tc_tiling_on_sc=False)


@functools.partial(
    pl.kernel,
    out_type=jax.ShapeDtypeStruct((NC, 1, N), jnp.float32),
    mesh=_vmesh,
    scratch_types=[
        pltpu.VMEM((NCHUNK, K), jnp.int32),
        pltpu.VMEM((NCHUNK, K), jnp.float32),
        pltpu.VMEM((N,), jnp.float32),
        pltpu.VMEM_SHARED((N,), jnp.float32),
        pltpu.SemaphoreType.DMA,
    ],
    compiler_params=_sc_params,
)
def _sc_deg(col_hbm, ew_hbm, out_hbm, col_v, ew_v, zb_v, deg_sp, sem):
    c = lax.axis_index("c")
    s = lax.axis_index("s")
    w = c * NS + s

    @pl.when(s == 0)
    def _():
        zeros16 = jnp.zeros((16,), jnp.float32)

        @pl.loop(0, N // 16)
        def _(j):
            zb_v[pl.ds(pl.multiple_of(j * 16, 16), 16)] = zeros16

        pltpu.sync_copy(zb_v, deg_sp)

    pltpu.sync_copy(col_hbm.at[w], col_v)
    pltpu.sync_copy(ew_hbm.at[w], ew_v)
    plsc.subcore_barrier()

    @pl.loop(0, NCHUNK)
    def _(j):
        pltpu.sync_copy(ew_v.at[j], deg_sp.at[col_v.at[j]], add=True)

    plsc.subcore_barrier()

    @pl.when(s == 0)
    def _():
        pltpu.async_copy(deg_sp, out_hbm.at[c, 0], sem).wait()


_acc_half_type = jax.ShapeDtypeStruct((NC, N, DH), jnp.float32)


@functools.partial(
    pl.kernel,
    out_type=[_acc_half_type, _acc_half_type],
    mesh=_vmesh,
    scratch_types=[
        pltpu.VMEM((NCHUNK, K), jnp.int32),
        pltpu.VMEM((NCHUNK, K), jnp.int32),
        pltpu.VMEM((NCHUNK, K), jnp.float32),
        pltpu.VMEM((K, DH), jnp.float32),
        pltpu.VMEM((K, DH), jnp.float32),
        pltpu.VMEM((K, DH), jnp.float32),
        pltpu.VMEM((K, DH), jnp.float32),
        pltpu.VMEM((ZCH, DH), jnp.float32),
        pltpu.VMEM_SHARED((N, DH), jnp.float32),
        pltpu.SemaphoreType.DMA,
        pltpu.SemaphoreType.DMA,
        pltpu.SemaphoreType.DMA,
        pltpu.SemaphoreType.DMA,
        pltpu.SemaphoreType.DMA,
    ],
    compiler_params=_sc_params,
)
def _sc_scatter(ga_hbm, gb_hbm, row_hbm, col_hbm, ew_hbm, outa_hbm, outb_hbm,
                row_v, col_v, ew_v, rows0_v, rows1_v, sc0_v, sc1_v, zb_v,
                acc_sp, sem, sem0, sem1, sems0, sems1):
    c = lax.axis_index("c")
    s = lax.axis_index("s")
    w = c * NS + s

    zeros16 = jnp.zeros((16,), jnp.float32)

    @pl.loop(0, ZCH)
    def _(j):
        for f in range(DH // 16):
            zb_v[j, pl.ds(f * 16, 16)] = zeros16

    pltpu.sync_copy(row_hbm.at[w], row_v)
    pltpu.sync_copy(col_hbm.at[w], col_v)
    pltpu.sync_copy(ew_hbm.at[w], ew_v)

    for p, (g_hbm, out_hbm) in enumerate(((ga_hbm, outa_hbm),
                                          (gb_hbm, outb_hbm))):
        @pl.loop(0, pl.cdiv(NZ, NS))
        def _(t):
            m = s + t * NS

            @pl.when(m < NZ)
            def _():
                off = pl.multiple_of(m * ZCH, 8)
                pltpu.sync_copy(zb_v, acc_sp.at[pl.ds(off, ZCH)])

        plsc.subcore_barrier()

        def _wait_g(buf, sg, j):
            pltpu.make_async_copy(g_hbm.at[row_v.at[j]], buf, sg).wait()

        def _wait_s(buf, ss, j):
            pltpu.make_async_copy(buf, acc_sp.at[col_v.at[j]], ss).wait()

        def _mul(j, src, dst):
            @plsc.parallel_loop(0, K, step=16)
            def _(k0):
                ew16 = ew_v[j, pl.ds(k0, 16)]
                for u in range(16):
                    sc = ew16[u]
                    for f in range(DH // 16):
                        sl = pl.ds(f * 16, 16)
                        dst[k0 + u, sl] = src[k0 + u, sl] * sc

        pltpu.async_copy(g_hbm.at[row_v.at[0]], rows0_v, sem0)

        @pl.loop(0, (NCHUNK - 1) // 2)
        def _(jj):
            j0 = jj * 2
            _wait_g(rows0_v, sem0, j0)
            pltpu.async_copy(g_hbm.at[row_v.at[j0 + 1]], rows1_v, sem1)

            @pl.when(jj > 0)
            def _():
                _wait_s(sc0_v, sems0, j0 - 2)
                _wait_s(sc1_v, sems1, j0 - 1)

            _mul(j0, rows0_v, sc0_v)
            pltpu.async_copy(sc0_v, acc_sp.at[col_v.at[j0]], sems0, add=True)
            _wait_g(rows1_v, sem1, j0 + 1)
            pltpu.async_copy(g_hbm.at[row_v.at[j0 + 2]], rows0_v, sem0)
            _mul(j0 + 1, rows1_v, sc1_v)
            pltpu.async_copy(sc1_v, acc_sp.at[col_v.at[j0 + 1]], sems1,
                             add=True)

        _wait_g(rows0_v, sem0, NCHUNK - 1)
        _wait_s(sc0_v, sems0, NCHUNK - 3)
        _mul(NCHUNK - 1, rows0_v, sc0_v)
        pltpu.async_copy(sc0_v, acc_sp.at[col_v.at[NCHUNK - 1]], sems0,
                         add=True)
        _wait_s(sc1_v, sems1, NCHUNK - 2)
        _wait_s(sc0_v, sems0, NCHUNK - 1)

        plsc.subcore_barrier()

        @pl.loop(0, pl.cdiv(NZ, NS))
        def _(t):
            m = s + t * NS

            @pl.when(m < NZ)
            def _():
                off = pl.multiple_of(m * ZCH, 8)
                pltpu.async_copy(acc_sp.at[pl.ds(off, ZCH)],
                                 out_hbm.at[c, pl.ds(off, ZCH)], sem)

        @pl.loop(0, pl.cdiv(NZ, NS))
        def _(t):
            m = s + t * NS

            @pl.when(m < NZ)
            def _():
                off = pl.multiple_of(m * ZCH, 8)
                pltpu.make_async_copy(acc_sp.at[pl.ds(off, ZCH)],
                                      out_hbm.at[c, pl.ds(off, ZCH)],
                                      sem).wait()

        plsc.subcore_barrier()


_BLK = 1000
_GRID = N // _BLK


def _mm_body(x_ref, w_ref, o_ref):
    o_ref[...] = jnp.dot(x_ref[...], w_ref[...],
                         preferred_element_type=jnp.float32)


def _tc_matmul(x, w):
    return pl.pallas_call(
        _mm_body,
        grid=(_GRID,),
        in_specs=[pl.BlockSpec((_BLK, D), lambda i: (i, 0)),
                  pl.BlockSpec((D, D), lambda i: (0, 0))],
        out_specs=pl.BlockSpec((_BLK, D), lambda i: (i, 0)),
        out_shape=jax.ShapeDtypeStruct((N, D), jnp.float32),
    )(x, w)


def _scale_body(degp_ref, h_ref, dinv_ref, ga_ref, gb_ref):
    deg = degp_ref[0] + degp_ref[1] + 1.0
    dv = lax.rsqrt(deg)
    dinv_ref[...] = dv
    g = dv * h_ref[...]
    ga_ref[...] = g[:, :DH]
    gb_ref[...] = g[:, DH:]


def _tc_scale(degp, h):
    return pl.pallas_call(
        _scale_body,
        grid=(_GRID,),
        in_specs=[pl.BlockSpec((NC, _BLK, 1), lambda i: (0, i, 0)),
                  pl.BlockSpec((_BLK, D), lambda i: (i, 0))],
        out_specs=[pl.BlockSpec((_BLK, 1), lambda i: (i, 0)),
                   pl.BlockSpec((_BLK, DH), lambda i: (i, 0)),
                   pl.BlockSpec((_BLK, DH), lambda i: (i, 0))],
        out_shape=[jax.ShapeDtypeStruct((N, 1), jnp.float32),
                   jax.ShapeDtypeStruct((N, DH), jnp.float32),
                   jax.ShapeDtypeStruct((N, DH), jnp.float32)],
    )(degp, h)


def _combine(acca_ref, accb_ref, ga_ref, gb_ref):
    ta = acca_ref[0] + acca_ref[1] + ga_ref[...]
    tb = accb_ref[0] + accb_ref[1] + gb_ref[...]
    return jnp.concatenate([ta, tb], axis=1)


def _post_mm_body(acca_ref, accb_ref, ga_ref, gb_ref, dinv_ref, b_ref, w_ref,
                  oa_ref, ob_ref):
    dv = dinv_ref[...]
    t = _combine(acca_ref, accb_ref, ga_ref, gb_ref)
    t = jnp.maximum(dv * t + b_ref[...], 0.0)
    r = dv * jnp.dot(t, w_ref[...], preferred_element_type=jnp.float32)
    oa_ref[...] = r[:, :DH]
    ob_ref[...] = r[:, DH:]


def _tc_post_mm(acca, accb, ga, gb, dinv, b, w):
    return pl.pallas_call(
        _post_mm_body,
        grid=(_GRID,),
        in_specs=[pl.BlockSpec((NC, _BLK, DH), lambda i: (0, i, 0)),
                  pl.BlockSpec((NC, _BLK, DH), lambda i: (0, i, 0)),
                  pl.BlockSpec((_BLK, DH), lambda i: (i, 0)),
                  pl.BlockSpec((_BLK, DH), lambda i: (i, 0)),
                  pl.BlockSpec((_BLK, 1), lambda i: (i, 0)),
                  pl.BlockSpec((1, D), lambda i: (0, 0)),
                  pl.BlockSpec((D, D), lambda i: (0, 0))],
        out_specs=[pl.BlockSpec((_BLK, DH), lambda i: (i, 0)),
                   pl.BlockSpec((_BLK, DH), lambda i: (i, 0))],
        out_shape=[jax.ShapeDtypeStruct((N, DH), jnp.float32),
                   jax.ShapeDtypeStruct((N, DH), jnp.float32)],
    )(acca, accb, ga, gb, dinv, b, w)


def _final_body(acca_ref, accb_ref, ga_ref, gb_ref, dinv_ref, b_ref,
                batch_ref, fcw_ref, fcb_ref, o_ref, sums_ref, cnts_ref):
    i = pl.program_id(0)

    @pl.when(i == 0)
    def _():
        sums_ref[...] = jnp.zeros_like(sums_ref)
        cnts_ref[...] = jnp.zeros_like(cnts_ref)

    dv = dinv_ref[...]
    t = _combine(acca_ref, accb_ref, ga_ref, gb_ref)
    h3r = jnp.maximum(dv * t + b_ref[...], 0.0)
    gid = lax.broadcasted_iota(jnp.int32, (_BLK, NG), 1)
    oh = (batch_ref[...] == gid).astype(jnp.float32)
    dn = (((0,), (0,)), ((), ()))
    sums_ref[...] += lax.dot_general(oh, h3r, dn,
                                     preferred_element_type=jnp.float32)
    cnts_ref[...] += lax.dot_general(oh, jnp.ones_like(h3r), dn,
                                     preferred_element_type=jnp.float32)

    @pl.when(i == _GRID - 1)
    def _():
        pooled = sums_ref[...] / jnp.maximum(cnts_ref[...], 1.0)
        z = jnp.dot(pooled, fcw_ref[...],
                    preferred_element_type=jnp.float32) + fcb_ref[...]
        o_ref[...] = 1.0 / (1.0 + jnp.exp(-z))


def _tc_final(acca, accb, ga, gb, dinv, b, batch2, fc_w, fc_b):
    return pl.pallas_call(
        _final_body,
        grid=(_GRID,),
        in_specs=[pl.BlockSpec((NC, _BLK, DH), lambda i: (0, i, 0)),
                  pl.BlockSpec((NC, _BLK, DH), lambda i: (0, i, 0)),
                  pl.BlockSpec((_BLK, DH), lambda i: (i, 0)),
                  pl.BlockSpec((_BLK, DH), lambda i: (i, 0)),
                  pl.BlockSpec((_BLK, 1), lambda i: (i, 0)),
                  pl.BlockSpec((1, D), lambda i: (0, 0)),
                  pl.BlockSpec((_BLK, 1), lambda i: (i, 0)),
                  pl.BlockSpec((D, 1), lambda i: (0, 0)),
                  pl.BlockSpec((1, 1), lambda i: (0, 0))],
        out_specs=pl.BlockSpec((NG, 1), lambda i: (0, 0)),
        out_shape=jax.ShapeDtypeStruct((NG, 1), jnp.float32),
        scratch_shapes=[pltpu.VMEM((NG, D), jnp.float32),
                        pltpu.VMEM((NG, D), jnp.float32)],
    )(acca, accb, ga, gb, dinv, b, batch2, fc_w, fc_b)


def kernel(x, edge_index, edge_attr, batch, W1, b1, W2, b2, W3, b3, fc_w, fc_b):
    row = edge_index[0].astype(jnp.int32).reshape(NT, NCHUNK, K)
    col = edge_index[1].astype(jnp.int32).reshape(NT, NCHUNK, K)
    ew = edge_attr.astype(jnp.float32).reshape(NT, NCHUNK, K)
    batch2 = batch.astype(jnp.int32).reshape(N, 1)
    b1r = b1.reshape(1, D)
    b2r = b2.reshape(1, D)
    b3r = b3.reshape(1, D)
    fcb = fc_b.reshape(1, 1)

    degp = _sc_deg(col, ew)
    h1 = _tc_matmul(x, W1)
    dinv, g1a, g1b = _tc_scale(degp.reshape(NC, N, 1), h1)

    a1a, a1b = _sc_scatter(g1a, g1b, row, col, ew)
    g2a, g2b = _tc_post_mm(a1a, a1b, g1a, g1b, dinv, b1r, W2)
    a2a, a2b = _sc_scatter(g2a, g2b, row, col, ew)
    g3a, g3b = _tc_post_mm(a2a, a2b, g2a, g2b, dinv, b2r, W3)
    a3a, a3b = _sc_scatter(g3a, g3b, row, col, ew)
    return _tc_final(a3a, a3b, g3a, g3b, dinv, b3r, batch2, fc_w, fcb)
